# baseline (speedup 1.0000x reference)
import functools

import jax
import jax.numpy as jnp
from jax import lax
from jax.experimental import pallas as pl
from jax.experimental.pallas import tpu as pltpu
from jax.experimental.pallas import tpu_sc as plsc

N_FEAT = 128
N_NODES = 10000
N_EDGES = 320000
N_RBF = 20

_LOG2 = 0.6931471805599453


def _ssp(v):
    return jnp.maximum(v, 0.0) + jnp.log1p(jnp.exp(-jnp.abs(v))) - _LOG2


def _h_body(x_ref, w_ref, b_ref, o_ref):
    o_ref[...] = (
        jnp.dot(x_ref[...], w_ref[...], preferred_element_type=jnp.float32)
        + b_ref[...]
    )


def _wij_body(f_ref, w1_ref, b1_ref, w2_ref, b2_ref, o_ref):
    w = jnp.dot(f_ref[...], w1_ref[...], preferred_element_type=jnp.float32)
    w = _ssp(w + b1_ref[...])
    o_ref[...] = (
        jnp.dot(w, w2_ref[...], preferred_element_type=jnp.float32)
        + b2_ref[...]
    )


def _out_body(pa_ref, pb_ref, w1_ref, b1_ref, w2_ref, b2_ref, o_ref):
    agg = (pa_ref[0] + pa_ref[1]) + (pb_ref[0] + pb_ref[1])
    o = jnp.dot(agg, w1_ref[...], preferred_element_type=jnp.float32)
    o = _ssp(o + b1_ref[...])
    o_ref[...] = (
        jnp.dot(o, w2_ref[...], preferred_element_type=jnp.float32) + b2_ref[...]
    )


_NTILES = 32
_NSPLIT = 2
_EHALF = N_EDGES // _NSPLIT
_EPT = _EHALF // _NTILES
_C = 40
_NCHUNK = _EPT // _C
_SLAB = 40
_NSLAB = N_NODES // _SLAB


def _sc_body(idx_off, h_hbm, wij_hbm, idxj_hbm, idxi_hbm, rc_hbm, out_hbm,
             idxj0_v, idxj1_v, idxi0_v, idxi1_v,
             rows0_v, rows1_v, wij0_v, wij1_v, rc0_v, rc1_v, acc_sh,
             jsem0, jsem1, msem0, msem1,
             gsem0, gsem1, wsem0, wsem1, rsem0, rsem1, ssem0, ssem1):
    core = lax.axis_index("c")
    sub = lax.axis_index("s")
    tile_base = pl.multiple_of((sub * 2 + core) * _EPT, _EPT)

    def zrow(r, _):
        for cb in range(N_FEAT // 16):
            rows0_v[r, pl.ds(cb * 16, 16)] = jnp.zeros((16,), jnp.float32)
        return 0
    lax.fori_loop(0, _SLAB, zrow, 0)
    for t in range((_NSLAB + 15) // 16):
        sl = sub + 16 * t

        @pl.when(sl < _NSLAB)
        def _():
            off = pl.multiple_of(sl * _SLAB, _SLAB)
            pltpu.sync_copy(rows0_v, acc_sh.at[pl.ds(off, _SLAB)])
    plsc.subcore_barrier()

    idxj = (idxj0_v, idxj1_v)
    idxi = (idxi0_v, idxi1_v)
    rows = (rows0_v, rows1_v)
    wijb = (wij0_v, wij1_v)
    rcb = (rc0_v, rc1_v)
    jsem = (jsem0, jsem1)
    msem = (msem0, msem1)
    gsem = (gsem0, gsem1)
    wsem = (wsem0, wsem1)
    rsem = (rsem0, rsem1)
    ssem = (ssem0, ssem1)

    def mul(rv, wv, rcv):
        for gb, rlo, rhi in ((0, 0, 16), (16, 16, 32), (_C - 16, 32, _C)):
            v = rcv[pl.ds(gb, 16)]
            for r in range(rlo, rhi):
                rc = v[r - gb]
                for cb in range(N_FEAT // 16):
                    sl = pl.ds(cb * 16, 16)
                    rv[r, sl] = rv[r, sl] * wv[r, sl] * rc

    def do_pair(k0, nb):
        dj, di, dw, dr = [], [], [], []
        for b in range(nb):
            base = pl.multiple_of(tile_base + (k0 + b) * _C, _C)
            dj.append(pltpu.async_copy(
                idxj_hbm.at[pl.ds(base + idx_off, _C)], idxj[b], jsem[b]))
            di.append(pltpu.async_copy(
                idxi_hbm.at[pl.ds(base + idx_off, _C)], idxi[b], msem[b]))
            dw.append(pltpu.async_copy(wij_hbm.at[pl.ds(base, _C)], wijb[b],
                                       wsem[b]))
            dr.append(pltpu.async_copy(
                rc_hbm.at[pl.ds(base + idx_off, _C)], rcb[b], rsem[b]))
        dg = []
        for b in range(nb):
            dj[b].wait()
            dg.append(pltpu.async_copy(h_hbm.at[idxj[b]], rows[b], gsem[b]))
        dsc = []
        for b in range(nb):
            dg[b].wait()
            dw[b].wait()
            dr[b].wait()
            mul(rows[b], wijb[b], rcb[b])
            di[b].wait()
            dsc.append(pltpu.async_copy(rows[b], acc_sh.at[idxi[b]],
                                        ssem[b], add=True))
        for b in range(nb):
            dsc[b].wait()

    def pair(g, _):
        do_pair(g * 2, 2)
        return 0
    lax.fori_loop(0, _NCHUNK // 2, pair, 0)
    if _NCHUNK % 2:
        do_pair(_NCHUNK - 1, 1)

    plsc.subcore_barrier()
    for t in range((_NSLAB + 15) // 16):
        sl = sub + 16 * t

        @pl.when(sl < _NSLAB)
        def _():
            off = pl.multiple_of(sl * _SLAB, _SLAB)
            pltpu.sync_copy(acc_sh.at[pl.ds(off, _SLAB)],
                            out_hbm.at[core, pl.ds(off, _SLAB)])


@functools.cache
def _sc_edge_stage(idx_off):
    return pl.kernel(
        functools.partial(_sc_body, idx_off),
        out_type=jax.ShapeDtypeStruct((2, N_NODES, N_FEAT), jnp.float32),
        mesh=plsc.VectorSubcoreMesh(core_axis_name="c", subcore_axis_name="s"),
        scratch_types=[
            pltpu.VMEM((_C,), jnp.int32),
            pltpu.VMEM((_C,), jnp.int32),
            pltpu.VMEM((_C,), jnp.int32),
            pltpu.VMEM((_C,), jnp.int32),
            pltpu.VMEM((_C, N_FEAT), jnp.float32),
            pltpu.VMEM((_C, N_FEAT), jnp.float32),
            pltpu.VMEM((_C, N_FEAT), jnp.float32),
            pltpu.VMEM((_C, N_FEAT), jnp.float32),
            pltpu.VMEM((_C,), jnp.float32),
            pltpu.VMEM((_C,), jnp.float32),
            pltpu.VMEM_SHARED((N_NODES, N_FEAT), jnp.float32),
        ] + [pltpu.SemaphoreType.DMA] * 12,
    )


def kernel(x, f_ij, idx_i, idx_j, rcut_ij,
           W_in, b_in, W_f1, b_f1, W_f2, b_f2,
           W_o1, b_o1, W_o2, b_o2):
    batch, atoms, feat = x.shape
    x2 = x.reshape(batch * atoms, feat)

    mb = 2000
    h = pl.pallas_call(
        _h_body,
        grid=(N_NODES // mb,),
        in_specs=[
            pl.BlockSpec((mb, feat), lambda i: (i, 0)),
            pl.BlockSpec((feat, N_FEAT), lambda i: (0, 0)),
            pl.BlockSpec((1, N_FEAT), lambda i: (0, 0)),
        ],
        out_specs=pl.BlockSpec((mb, N_FEAT), lambda i: (i, 0)),
        out_shape=jax.ShapeDtypeStruct((N_NODES, N_FEAT), jnp.float32),
    )(x2, W_in, b_in.reshape(1, N_FEAT))

    eb = 3200
    nblk = _EHALF // eb
    idx_j32 = idx_j.astype(jnp.int32)
    idx_i32 = idx_i.astype(jnp.int32)

    def wij_half(s):
        return pl.pallas_call(
            _wij_body,
            grid=(nblk,),
            in_specs=[
                pl.BlockSpec((eb, N_RBF), lambda i, s0=s * nblk: (s0 + i, 0)),
                pl.BlockSpec((N_RBF, N_FEAT), lambda i: (0, 0)),
                pl.BlockSpec((1, N_FEAT), lambda i: (0, 0)),
                pl.BlockSpec((N_FEAT, N_FEAT), lambda i: (0, 0)),
                pl.BlockSpec((1, N_FEAT), lambda i: (0, 0)),
            ],
            out_specs=pl.BlockSpec((eb, N_FEAT), lambda i: (i, 0)),
            out_shape=jax.ShapeDtypeStruct((_EHALF, N_FEAT), jnp.float32),
        )(f_ij, W_f1, b_f1.reshape(1, N_FEAT), W_f2, b_f2.reshape(1, N_FEAT))

    wij_a = wij_half(0)
    wij_b = wij_half(1)
    part_a = _sc_edge_stage(0)(h, wij_a, idx_j32, idx_i32, rcut_ij)
    part_b = _sc_edge_stage(_EHALF)(h, wij_b, idx_j32, idx_i32, rcut_ij)

    ob = 2000
    out = pl.pallas_call(
        _out_body,
        grid=(N_NODES // ob,),
        in_specs=[
            pl.BlockSpec((2, ob, N_FEAT), lambda i: (0, i, 0)),
            pl.BlockSpec((2, ob, N_FEAT), lambda i: (0, i, 0)),
            pl.BlockSpec((N_FEAT, N_FEAT), lambda i: (0, 0)),
            pl.BlockSpec((1, N_FEAT), lambda i: (0, 0)),
            pl.BlockSpec((N_FEAT, N_FEAT), lambda i: (0, 0)),
            pl.BlockSpec((1, N_FEAT), lambda i: (0, 0)),
        ],
        out_specs=pl.BlockSpec((ob, N_FEAT), lambda i: (i, 0)),
        out_shape=jax.ShapeDtypeStruct((N_NODES, N_FEAT), jnp.float32),
    )(part_a, part_b, W_o1, b_o1.reshape(1, N_FEAT),
      W_o2, b_o2.reshape(1, N_FEAT))

    return out.reshape(batch, atoms, N_FEAT)


# speedup vs baseline: 1.7965x; 1.7965x over previous
import functools

import jax
import jax.numpy as jnp
from jax import lax
from jax.experimental import pallas as pl
from jax.experimental.pallas import tpu as pltpu
from jax.experimental.pallas import tpu_sc as plsc

N_FEAT = 128
N_NODES = 10000
N_EDGES = 320000
N_RBF = 20

_LOG2 = 0.6931471805599453


def _ssp(v):
    return jnp.maximum(v, 0.0) + jnp.log1p(jnp.exp(-jnp.abs(v))) - _LOG2


def _h_body(x_ref, w_ref, b_ref, o_ref):
    o_ref[...] = (
        jnp.dot(x_ref[...], w_ref[...], preferred_element_type=jnp.float32)
        + b_ref[...]
    )


def _wij_body(s0, f_ref, rc_ref, w1_ref, b1_ref, w2_ref, b2_ref, o_ref):
    w = lax.dot_general(f_ref[...], w1_ref[...], (((0,), (0,)), ((), ())),
                        preferred_element_type=jnp.float32)
    w = _ssp(w + b1_ref[...])
    w = jnp.dot(w, w2_ref[...], preferred_element_type=jnp.float32) + b2_ref[...]
    eb = w.shape[0]
    rb = eb // 128
    rc = rc_ref[pl.ds((s0 + pl.program_id(0)) * rb, rb), :]
    w3 = w.reshape(rb, 128, N_FEAT)
    rc3 = lax.broadcast_in_dim(rc, (rb, 128, N_FEAT), (0, 1))
    o_ref[...] = (w3 * rc3).reshape(eb, N_FEAT)


def _out_body(pa_ref, pb_ref, w1_ref, b1_ref, w2_ref, b2_ref, o_ref):
    agg = (pa_ref[0] + pa_ref[1]) + (pb_ref[0] + pb_ref[1])
    o = jnp.dot(agg, w1_ref[...], preferred_element_type=jnp.float32)
    o = _ssp(o + b1_ref[...])
    o_ref[...] = (
        jnp.dot(o, w2_ref[...], preferred_element_type=jnp.float32) + b2_ref[...]
    )


_NTILES = 32
_NSPLIT = 2
_EHALF = N_EDGES // _NSPLIT
_EPT = _EHALF // _NTILES
_C = 40
_NCHUNK = _EPT // _C
_SLAB = 40
_NSLAB = N_NODES // _SLAB


def _sc_body(idx_off, h_hbm, wij_hbm, idxj_hbm, idxi_hbm, out_hbm,
             idxj0_v, idxj1_v, idxi0_v, idxi1_v,
             rows0_v, rows1_v, wij0_v, wij1_v, acc_sh,
             jsem0, jsem1, msem0, msem1,
             gsem0, gsem1, wsem0, wsem1, ssem0, ssem1):
    core = lax.axis_index("c")
    sub = lax.axis_index("s")
    tile_base = pl.multiple_of((sub * 2 + core) * _EPT, _EPT)

    def zrow(r, _):
        for cb in range(N_FEAT // 16):
            rows0_v[r, pl.ds(cb * 16, 16)] = jnp.zeros((16,), jnp.float32)
        return 0
    lax.fori_loop(0, _SLAB, zrow, 0)
    for t in range((_NSLAB + 15) // 16):
        sl = sub + 16 * t

        @pl.when(sl < _NSLAB)
        def _():
            off = pl.multiple_of(sl * _SLAB, _SLAB)
            pltpu.sync_copy(rows0_v, acc_sh.at[pl.ds(off, _SLAB)])
    plsc.subcore_barrier()

    idxj = (idxj0_v, idxj1_v)
    idxi = (idxi0_v, idxi1_v)
    rows = (rows0_v, rows1_v)
    wijb = (wij0_v, wij1_v)
    jsem = (jsem0, jsem1)
    msem = (msem0, msem1)
    gsem = (gsem0, gsem1)
    wsem = (wsem0, wsem1)
    ssem = (ssem0, ssem1)

    def mul(rv, wv):
        def body(r, _):
            for cb in range(N_FEAT // 16):
                sl = pl.ds(cb * 16, 16)
                rv[r, sl] = rv[r, sl] * wv[r, sl]
            return 0
        lax.fori_loop(0, _C, body, 0)

    def do_pair(k0, nb):
        dj, di, dw = [], [], []
        for b in range(nb):
            base = pl.multiple_of(tile_base + (k0 + b) * _C, _C)
            dj.append(pltpu.async_copy(
                idxj_hbm.at[pl.ds(base + idx_off, _C)], idxj[b], jsem[b]))
            di.append(pltpu.async_copy(
                idxi_hbm.at[pl.ds(base + idx_off, _C)], idxi[b], msem[b]))
            dw.append(pltpu.async_copy(wij_hbm.at[pl.ds(base, _C)], wijb[b],
                                       wsem[b]))
        dg = []
        for b in range(nb):
            dj[b].wait()
            dg.append(pltpu.async_copy(h_hbm.at[idxj[b]], rows[b], gsem[b]))
        dsc = []
        for b in range(nb):
            dg[b].wait()
            dw[b].wait()
            mul(rows[b], wijb[b])
            di[b].wait()
            dsc.append(pltpu.async_copy(rows[b], acc_sh.at[idxi[b]],
                                        ssem[b], add=True))
        for b in range(nb):
            dsc[b].wait()

    def pair(g, _):
        do_pair(g * 2, 2)
        return 0
    lax.fori_loop(0, _NCHUNK // 2, pair, 0)
    if _NCHUNK % 2:
        do_pair(_NCHUNK - 1, 1)

    plsc.subcore_barrier()
    for t in range((_NSLAB + 15) // 16):
        sl = sub + 16 * t

        @pl.when(sl < _NSLAB)
        def _():
            off = pl.multiple_of(sl * _SLAB, _SLAB)
            pltpu.sync_copy(acc_sh.at[pl.ds(off, _SLAB)],
                            out_hbm.at[core, pl.ds(off, _SLAB)])


@functools.cache
def _sc_edge_stage(idx_off):
    return pl.kernel(
        functools.partial(_sc_body, idx_off),
        out_type=jax.ShapeDtypeStruct((2, N_NODES, N_FEAT), jnp.float32),
        mesh=plsc.VectorSubcoreMesh(core_axis_name="c", subcore_axis_name="s"),
        scratch_types=[
            pltpu.VMEM((_C,), jnp.int32),
            pltpu.VMEM((_C,), jnp.int32),
            pltpu.VMEM((_C,), jnp.int32),
            pltpu.VMEM((_C,), jnp.int32),
            pltpu.VMEM((_C, N_FEAT), jnp.float32),
            pltpu.VMEM((_C, N_FEAT), jnp.float32),
            pltpu.VMEM((_C, N_FEAT), jnp.float32),
            pltpu.VMEM((_C, N_FEAT), jnp.float32),
            pltpu.VMEM_SHARED((N_NODES, N_FEAT), jnp.float32),
        ] + [pltpu.SemaphoreType.DMA] * 10,
    )


def kernel(x, f_ij, idx_i, idx_j, rcut_ij,
           W_in, b_in, W_f1, b_f1, W_f2, b_f2,
           W_o1, b_o1, W_o2, b_o2):
    batch, atoms, feat = x.shape
    x2 = x.reshape(batch * atoms, feat)

    mb = 2000
    h = pl.pallas_call(
        _h_body,
        grid=(N_NODES // mb,),
        in_specs=[
            pl.BlockSpec((mb, feat), lambda i: (i, 0)),
            pl.BlockSpec((feat, N_FEAT), lambda i: (0, 0)),
            pl.BlockSpec((1, N_FEAT), lambda i: (0, 0)),
        ],
        out_specs=pl.BlockSpec((mb, N_FEAT), lambda i: (i, 0)),
        out_shape=jax.ShapeDtypeStruct((N_NODES, N_FEAT), jnp.float32),
    )(x2, W_in, b_in.reshape(1, N_FEAT))

    eb = 3200
    nblk = _EHALF // eb
    rb = eb // 128
    idx_j32 = idx_j.astype(jnp.int32)
    idx_i32 = idx_i.astype(jnp.int32)
    f_t = f_ij.T
    rc2d = rcut_ij.reshape(-1, 128)

    def wij_half(s):
        return pl.pallas_call(
            functools.partial(_wij_body, s * nblk),
            grid=(nblk,),
            in_specs=[
                pl.BlockSpec((N_RBF, eb), lambda i, s0=s * nblk: (0, s0 + i)),
                pl.BlockSpec(rc2d.shape, lambda i: (0, 0)),
                pl.BlockSpec((N_RBF, N_FEAT), lambda i: (0, 0)),
                pl.BlockSpec((1, N_FEAT), lambda i: (0, 0)),
                pl.BlockSpec((N_FEAT, N_FEAT), lambda i: (0, 0)),
                pl.BlockSpec((1, N_FEAT), lambda i: (0, 0)),
            ],
            out_specs=pl.BlockSpec((eb, N_FEAT), lambda i: (i, 0)),
            out_shape=jax.ShapeDtypeStruct((_EHALF, N_FEAT), jnp.float32),
        )(f_t, rc2d, W_f1, b_f1.reshape(1, N_FEAT),
          W_f2, b_f2.reshape(1, N_FEAT))

    wij_a = wij_half(0)
    wij_b = wij_half(1)
    part_a = _sc_edge_stage(0)(h, wij_a, idx_j32, idx_i32)
    part_b = _sc_edge_stage(_EHALF)(h, wij_b, idx_j32, idx_i32)

    ob = 2000
    out = pl.pallas_call(
        _out_body,
        grid=(N_NODES // ob,),
        in_specs=[
            pl.BlockSpec((2, ob, N_FEAT), lambda i: (0, i, 0)),
            pl.BlockSpec((2, ob, N_FEAT), lambda i: (0, i, 0)),
            pl.BlockSpec((N_FEAT, N_FEAT), lambda i: (0, 0)),
            pl.BlockSpec((1, N_FEAT), lambda i: (0, 0)),
            pl.BlockSpec((N_FEAT, N_FEAT), lambda i: (0, 0)),
            pl.BlockSpec((1, N_FEAT), lambda i: (0, 0)),
        ],
        out_specs=pl.BlockSpec((ob, N_FEAT), lambda i: (i, 0)),
        out_shape=jax.ShapeDtypeStruct((N_NODES, N_FEAT), jnp.float32),
    )(part_a, part_b, W_o1, b_o1.reshape(1, N_FEAT),
      W_o2, b_o2.reshape(1, N_FEAT))

    return out.reshape(batch, atoms, N_FEAT)


# speedup vs baseline: 1.7991x; 1.0015x over previous
import functools

import jax
import jax.numpy as jnp
from jax import lax
from jax.experimental import pallas as pl
from jax.experimental.pallas import tpu as pltpu
from jax.experimental.pallas import tpu_sc as plsc

N_FEAT = 128
N_NODES = 10000
N_EDGES = 320000
N_RBF = 20

_LOG2 = 0.6931471805599453


def _ssp(v):
    return jnp.maximum(v, 0.0) + jnp.log1p(jnp.exp(-jnp.abs(v))) - _LOG2


def _h_body(x_ref, w_ref, b_ref, o_ref):
    o_ref[...] = (
        jnp.dot(x_ref[...], w_ref[...], preferred_element_type=jnp.float32)
        + b_ref[...]
    )


def _wij_body(s0, f_ref, rc_ref, w1_ref, b1_ref, w2_ref, b2_ref, o_ref):
    w = lax.dot_general(f_ref[...], w1_ref[...], (((0,), (0,)), ((), ())),
                        preferred_element_type=jnp.float32)
    w = _ssp(w + b1_ref[...])
    w = jnp.dot(w, w2_ref[...], preferred_element_type=jnp.float32) + b2_ref[...]
    eb = w.shape[0]
    rb = eb // 128
    rc = rc_ref[pl.ds((s0 + pl.program_id(0)) * rb, rb), :]
    w3 = w.reshape(rb, 128, N_FEAT)
    rc3 = lax.broadcast_in_dim(rc, (rb, 128, N_FEAT), (0, 1))
    o_ref[...] = (w3 * rc3).reshape(eb, N_FEAT)


def _out_body(pa_ref, pb_ref, w1_ref, b1_ref, w2_ref, b2_ref, o_ref):
    agg = (pa_ref[0] + pa_ref[1]) + (pb_ref[0] + pb_ref[1])
    o = jnp.dot(agg, w1_ref[...], preferred_element_type=jnp.float32)
    o = _ssp(o + b1_ref[...])
    o_ref[...] = (
        jnp.dot(o, w2_ref[...], preferred_element_type=jnp.float32) + b2_ref[...]
    )


_NTILES = 32
_NSPLIT = 2
_EHALF = N_EDGES // _NSPLIT
_EPT = _EHALF // _NTILES
_C = 40
_NCHUNK = _EPT // _C
_SLAB = 40
_NSLAB = N_NODES // _SLAB


def _sc_body(idx_off, h_hbm, wij_hbm, idxj_hbm, idxi_hbm, out_hbm,
             idxj0_v, idxj1_v, idxi0_v, idxi1_v,
             rows0_v, rows1_v, wij0_v, wij1_v, acc_sh,
             jsem0, jsem1, msem0, msem1,
             gsem0, gsem1, wsem0, wsem1, ssem0, ssem1):
    core = lax.axis_index("c")
    sub = lax.axis_index("s")
    tile_base = pl.multiple_of((sub * 2 + core) * _EPT, _EPT)

    def zrow(r, _):
        for cb in range(N_FEAT // 16):
            rows0_v[r, pl.ds(cb * 16, 16)] = jnp.zeros((16,), jnp.float32)
        return 0
    lax.fori_loop(0, _SLAB, zrow, 0)
    for t in range((_NSLAB + 15) // 16):
        sl = sub + 16 * t

        @pl.when(sl < _NSLAB)
        def _():
            off = pl.multiple_of(sl * _SLAB, _SLAB)
            pltpu.sync_copy(rows0_v.at[pl.ds(0, _SLAB)],
                            acc_sh.at[pl.ds(off, _SLAB)])
    plsc.subcore_barrier()

    idxj = (idxj0_v, idxj1_v)
    idxi = (idxi0_v, idxi1_v)
    rows = (rows0_v, rows1_v)
    wijb = (wij0_v, wij1_v)
    jsem = (jsem0, jsem1)
    msem = (msem0, msem1)
    gsem = (gsem0, gsem1)
    wsem = (wsem0, wsem1)
    ssem = (ssem0, ssem1)

    def mul(rv, wv):
        def body(r, _):
            for cb in range(N_FEAT // 16):
                sl = pl.ds(cb * 16, 16)
                rv[r, sl] = rv[r, sl] * wv[r, sl]
            return 0
        lax.fori_loop(0, _C, body, 0)

    def drain_scatter(b):
        pltpu.make_async_copy(rows[b], acc_sh.at[idxi[b]], ssem[b]).wait()

    def do_pair(k0, nb, drain):
        for b in range(nb):
            if drain:
                drain_scatter(b)
        dj, di, dw = [], [], []
        for b in range(nb):
            base = pl.multiple_of(tile_base + (k0 + b) * _C, _C)
            dj.append(pltpu.async_copy(
                idxj_hbm.at[pl.ds(base + idx_off, _C)], idxj[b], jsem[b]))
            di.append(pltpu.async_copy(
                idxi_hbm.at[pl.ds(base + idx_off, _C)], idxi[b], msem[b]))
            dw.append(pltpu.async_copy(wij_hbm.at[pl.ds(base, _C)], wijb[b],
                                       wsem[b]))
        dg = []
        for b in range(nb):
            dj[b].wait()
            dg.append(pltpu.async_copy(h_hbm.at[idxj[b]], rows[b], gsem[b]))
        for b in range(nb):
            dg[b].wait()
            dw[b].wait()
            mul(rows[b], wijb[b])
            di[b].wait()
            pltpu.async_copy(rows[b], acc_sh.at[idxi[b]], ssem[b], add=True)

    do_pair(0, 2, False)

    def pair(g, _):
        do_pair(g * 2, 2, True)
        return 0
    lax.fori_loop(1, _NCHUNK // 2, pair, 0)
    if _NCHUNK % 2:
        do_pair(_NCHUNK - 1, 1, True)
        drain_scatter(0)
        drain_scatter(1)
    else:
        drain_scatter(0)
        drain_scatter(1)

    plsc.subcore_barrier()
    for t in range((_NSLAB + 15) // 16):
        sl = sub + 16 * t

        @pl.when(sl < _NSLAB)
        def _():
            off = pl.multiple_of(sl * _SLAB, _SLAB)
            pltpu.sync_copy(acc_sh.at[pl.ds(off, _SLAB)],
                            out_hbm.at[core, pl.ds(off, _SLAB)])


@functools.cache
def _sc_edge_stage(idx_off):
    return pl.kernel(
        functools.partial(_sc_body, idx_off),
        out_type=jax.ShapeDtypeStruct((2, N_NODES, N_FEAT), jnp.float32),
        mesh=plsc.VectorSubcoreMesh(core_axis_name="c", subcore_axis_name="s"),
        scratch_types=[
            pltpu.VMEM((_C,), jnp.int32),
            pltpu.VMEM((_C,), jnp.int32),
            pltpu.VMEM((_C,), jnp.int32),
            pltpu.VMEM((_C,), jnp.int32),
            pltpu.VMEM((_C, N_FEAT), jnp.float32),
            pltpu.VMEM((_C, N_FEAT), jnp.float32),
            pltpu.VMEM((_C, N_FEAT), jnp.float32),
            pltpu.VMEM((_C, N_FEAT), jnp.float32),
            pltpu.VMEM_SHARED((N_NODES, N_FEAT), jnp.float32),
        ] + [pltpu.SemaphoreType.DMA] * 10,
    )


def kernel(x, f_ij, idx_i, idx_j, rcut_ij,
           W_in, b_in, W_f1, b_f1, W_f2, b_f2,
           W_o1, b_o1, W_o2, b_o2):
    batch, atoms, feat = x.shape
    x2 = x.reshape(batch * atoms, feat)

    mb = 2000
    h = pl.pallas_call(
        _h_body,
        grid=(N_NODES // mb,),
        in_specs=[
            pl.BlockSpec((mb, feat), lambda i: (i, 0)),
            pl.BlockSpec((feat, N_FEAT), lambda i: (0, 0)),
            pl.BlockSpec((1, N_FEAT), lambda i: (0, 0)),
        ],
        out_specs=pl.BlockSpec((mb, N_FEAT), lambda i: (i, 0)),
        out_shape=jax.ShapeDtypeStruct((N_NODES, N_FEAT), jnp.float32),
    )(x2, W_in, b_in.reshape(1, N_FEAT))

    eb = 3200
    nblk = _EHALF // eb
    rb = eb // 128
    idx_j32 = idx_j.astype(jnp.int32)
    idx_i32 = idx_i.astype(jnp.int32)
    f_t = f_ij.T
    rc2d = rcut_ij.reshape(-1, 128)

    def wij_half(s):
        return pl.pallas_call(
            functools.partial(_wij_body, s * nblk),
            grid=(nblk,),
            in_specs=[
                pl.BlockSpec((N_RBF, eb), lambda i, s0=s * nblk: (0, s0 + i)),
                pl.BlockSpec(rc2d.shape, lambda i: (0, 0)),
                pl.BlockSpec((N_RBF, N_FEAT), lambda i: (0, 0)),
                pl.BlockSpec((1, N_FEAT), lambda i: (0, 0)),
                pl.BlockSpec((N_FEAT, N_FEAT), lambda i: (0, 0)),
                pl.BlockSpec((1, N_FEAT), lambda i: (0, 0)),
            ],
            out_specs=pl.BlockSpec((eb, N_FEAT), lambda i: (i, 0)),
            out_shape=jax.ShapeDtypeStruct((_EHALF, N_FEAT), jnp.float32),
        )(f_t, rc2d, W_f1, b_f1.reshape(1, N_FEAT),
          W_f2, b_f2.reshape(1, N_FEAT))

    wij_a = wij_half(0)
    wij_b = wij_half(1)
    part_a = _sc_edge_stage(0)(h, wij_a, idx_j32, idx_i32)
    part_b = _sc_edge_stage(_EHALF)(h, wij_b, idx_j32, idx_i32)

    ob = 2000
    out = pl.pallas_call(
        _out_body,
        grid=(N_NODES // ob,),
        in_specs=[
            pl.BlockSpec((2, ob, N_FEAT), lambda i: (0, i, 0)),
            pl.BlockSpec((2, ob, N_FEAT), lambda i: (0, i, 0)),
            pl.BlockSpec((N_FEAT, N_FEAT), lambda i: (0, 0)),
            pl.BlockSpec((1, N_FEAT), lambda i: (0, 0)),
            pl.BlockSpec((N_FEAT, N_FEAT), lambda i: (0, 0)),
            pl.BlockSpec((1, N_FEAT), lambda i: (0, 0)),
        ],
        out_specs=pl.BlockSpec((ob, N_FEAT), lambda i: (i, 0)),
        out_shape=jax.ShapeDtypeStruct((N_NODES, N_FEAT), jnp.float32),
    )(part_a, part_b, W_o1, b_o1.reshape(1, N_FEAT),
      W_o2, b_o2.reshape(1, N_FEAT))

    return out.reshape(batch, atoms, N_FEAT)
